# Initial kernel scaffold; baseline (speedup 1.0000x reference)
import jax
import jax.numpy as jnp
from jax.experimental import pallas as pl


def kernel(x, edge_index, edge_weight, W, b):
    raise NotImplementedError("write your pallas kernel here")


# speedup vs baseline: 4.8937x; 4.8937x over previous
import functools

import jax
import jax.numpy as jnp
from jax import lax
from jax.experimental import pallas as pl
from jax.experimental.pallas import tpu as pltpu
from jax.experimental.pallas import tpu_sc as plsc

N = 10000
E = 320000
D = 128

NC = 2
NS = 16
NW = NC * NS

CH = 128
NCHUNK = E // CH
BASE_CHUNKS = NCHUNK // NW
EXTRA = NCHUNK % NW
NPAD = 10112
ROWS_PER_TILE = NPAD // NS

MM_BLK = 1000


def _mm_body(x_ref, w_ref, b_ref, o_ref):
    o_ref[...] = lax.dot_general(
        x_ref[...], w_ref[...],
        dimension_numbers=(((1,), (1,)), ((), ())),
        preferred_element_type=jnp.float32,
    ) + b_ref[...]


def _support_matmul(x, W, b2):
    return pl.pallas_call(
        _mm_body,
        grid=(N // MM_BLK,),
        in_specs=[
            pl.BlockSpec((MM_BLK, D), lambda i: (i, 0)),
            pl.BlockSpec((D, D), lambda i: (0, 0)),
            pl.BlockSpec((1, D), lambda i: (0, 0)),
        ],
        out_specs=pl.BlockSpec((MM_BLK, D), lambda i: (i, 0)),
        out_shape=jax.ShapeDtypeStruct((N, D), jnp.float32),
    )(x, W, b2)


def _edge_body(sup_hbm, src_hbm, dst_hbm, ew_hbm, zero_hbm, out_hbm,
               srcv, dstv, wv, rows, agg, sem):
    cid = lax.axis_index("c")
    sid = lax.axis_index("s")
    wid = sid * NC + cid

    r0 = sid * ROWS_PER_TILE
    pltpu.sync_copy(zero_hbm, agg.at[pl.ds(r0, ROWS_PER_TILE)])
    plsc.subcore_barrier()

    nchunks = BASE_CHUNKS + jnp.where(wid < EXTRA, 1, 0)

    def chunk_body(i, carry):
        base = (wid + i * NW) * CH
        pltpu.sync_copy(src_hbm.at[pl.ds(base, CH)], srcv)
        pltpu.sync_copy(dst_hbm.at[pl.ds(base, CH)], dstv)
        pltpu.sync_copy(ew_hbm.at[pl.ds(base, CH)], wv)
        pltpu.async_copy(sup_hbm.at[srcv], rows, sem).wait()

        def scale_body(g, c2):
            w16 = wv[pl.ds(g * 16, 16)]
            for l in range(16):
                w = w16[l]
                e = g * 16 + l
                for j in range(D // 16):
                    sl = pl.ds(j * 16, 16)
                    rows[e, sl] = rows[e, sl] * w
            return c2

        lax.fori_loop(0, CH // 16, scale_body, 0)
        pltpu.sync_copy(rows, agg.at[dstv], add=True)
        return carry

    lax.fori_loop(0, nchunks, chunk_body, 0)
    plsc.subcore_barrier()

    pltpu.sync_copy(agg.at[pl.ds(r0, ROWS_PER_TILE)],
                    out_hbm.at[cid, pl.ds(r0, ROWS_PER_TILE)])


_edge_kernel = functools.partial(
    pl.kernel,
    out_type=jax.ShapeDtypeStruct((NC, NPAD, D), jnp.float32),
    mesh=plsc.VectorSubcoreMesh(core_axis_name="c", subcore_axis_name="s"),
    scratch_types=[
        pltpu.VMEM((CH,), jnp.int32),
        pltpu.VMEM((CH,), jnp.int32),
        pltpu.VMEM((CH,), jnp.float32),
        pltpu.VMEM((CH, D), jnp.float32),
        pltpu.VMEM_SHARED((NPAD, D), jnp.float32),
        pltpu.SemaphoreType.DMA,
    ],
)(_edge_body)


def _comb_body(p_ref, o_ref):
    o_ref[...] = jnp.tanh(p_ref[0] + p_ref[1])


def _combine(partials):
    return pl.pallas_call(
        _comb_body,
        grid=(N // MM_BLK,),
        in_specs=[pl.BlockSpec((NC, MM_BLK, D), lambda i: (0, i, 0))],
        out_specs=pl.BlockSpec((MM_BLK, D), lambda i: (i, 0)),
        out_shape=jax.ShapeDtypeStruct((N, D), jnp.float32),
    )(partials)


def kernel(x, edge_index, edge_weight, W, b):
    dst = edge_index[0].astype(jnp.int32)
    src = edge_index[1].astype(jnp.int32)
    support = _support_matmul(x, W, b.reshape(1, D))
    zeros = jnp.zeros((ROWS_PER_TILE, D), jnp.float32)
    partials = _edge_kernel(support, src, dst, edge_weight, zeros)
    return _combine(partials)


# speedup vs baseline: 10.0006x; 2.0435x over previous
import functools

import jax
import jax.numpy as jnp
from jax import lax
from jax.experimental import pallas as pl
from jax.experimental.pallas import tpu as pltpu
from jax.experimental.pallas import tpu_sc as plsc

N = 10000
E = 320000
D = 128

NC = 2
NS = 16
NW = NC * NS

CH = 128
EPAD = 327680
CHW = EPAD // CH // NW
NPAD = 10112
ROWS_PER_TILE = NPAD // NS

MM_BLK = 1000


def _mm_body(x_ref, w_ref, b_ref, o_ref):
    o_ref[...] = lax.dot_general(
        x_ref[...], w_ref[...],
        dimension_numbers=(((1,), (1,)), ((), ())),
        preferred_element_type=jnp.float32,
    ) + b_ref[...]


def _support_matmul(x, W, b2):
    return pl.pallas_call(
        _mm_body,
        grid=(N // MM_BLK,),
        in_specs=[
            pl.BlockSpec((MM_BLK, D), lambda i: (i, 0)),
            pl.BlockSpec((D, D), lambda i: (0, 0)),
            pl.BlockSpec((1, D), lambda i: (0, 0)),
        ],
        out_specs=pl.BlockSpec((MM_BLK, D), lambda i: (i, 0)),
        out_shape=jax.ShapeDtypeStruct((N, D), jnp.float32),
    )(x, W, b2)


def _edge_body(sup_hbm, src_hbm, dst_hbm, ew_hbm, zero_hbm, out_hbm,
               s0, s1, s2, s3, d0, d1, d2, d3, w0, w1, w2, w3,
               rows0, rows1, agg,
               isem0, isem1, isem2, isem3, gsem0, gsem1):
    srcs = (s0, s1, s2, s3)
    dsts = (d0, d1, d2, d3)
    ws = (w0, w1, w2, w3)
    isems = (isem0, isem1, isem2, isem3)
    gsems = (gsem0, gsem1)
    rows = (rows0, rows1)

    cid = lax.axis_index("c")
    sid = lax.axis_index("s")
    wid = sid * NC + cid
    base = wid * CHW

    r0 = sid * ROWS_PER_TILE
    pltpu.sync_copy(zero_hbm, agg.at[pl.ds(r0, ROWS_PER_TILE)])
    plsc.subcore_barrier()

    def idx_copies(t, s):
        off = (base + t) * CH
        return (
            pltpu.make_async_copy(src_hbm.at[pl.ds(off, CH)], srcs[s], isems[s]),
            pltpu.make_async_copy(dst_hbm.at[pl.ds(off, CH)], dsts[s], isems[s]),
            pltpu.make_async_copy(ew_hbm.at[pl.ds(off, CH)], ws[s], isems[s]),
        )

    def start_idx(t, s):
        for c in idx_copies(t, s):
            c.start()

    def wait_idx(t, s):
        for c in idx_copies(t, s):
            c.wait()

    def gather(s, r):
        return pltpu.make_async_copy(sup_hbm.at[srcs[s]], rows[r], gsems[r])

    start_idx(0, 0)
    wait_idx(0, 0)
    start_idx(1, 1)
    gather(0, 0).start()

    def body(g, carry):
        for k in range(4):
            t = g * 4 + k
            kn = (k + 1) % 4
            last_g = CHW // 4 - 1

            def prefetch_gather():
                wait_idx(t + 1, kn)
                gather(kn, (k + 1) % 2).start()

            if k == 3:
                pl.when(g < last_g)(prefetch_gather)
            else:
                prefetch_gather()

            gather(k, k % 2).wait()

            def prefetch_idx():
                start_idx(t + 2, (k + 2) % 4)

            if k >= 2:
                pl.when(g < last_g)(prefetch_idx)
            else:
                prefetch_idx()

            cur = rows[k % 2]
            wv = ws[k]

            def scale_body(grp, c2):
                w16 = wv[pl.ds(grp * 16, 16)]
                for l in range(16):
                    w = w16[l]
                    e = grp * 16 + l
                    for j in range(D // 16):
                        sl = pl.ds(j * 16, 16)
                        cur[e, sl] = cur[e, sl] * w
                return c2

            lax.fori_loop(0, CH // 16, scale_body, 0)

            pltpu.sync_copy(cur, agg.at[dsts[k]], add=True)
        return carry

    lax.fori_loop(0, CHW // 4, body, 0)
    plsc.subcore_barrier()

    pltpu.sync_copy(agg.at[pl.ds(r0, ROWS_PER_TILE)],
                    out_hbm.at[cid, pl.ds(r0, ROWS_PER_TILE)])


_edge_kernel = functools.partial(
    pl.kernel,
    out_type=jax.ShapeDtypeStruct((NC, NPAD, D), jnp.float32),
    mesh=plsc.VectorSubcoreMesh(core_axis_name="c", subcore_axis_name="s"),
    scratch_types=(
        [pltpu.VMEM((CH,), jnp.int32)] * 4
        + [pltpu.VMEM((CH,), jnp.int32)] * 4
        + [pltpu.VMEM((CH,), jnp.float32)] * 4
        + [pltpu.VMEM((CH, D), jnp.float32)] * 2
        + [pltpu.VMEM_SHARED((NPAD, D), jnp.float32)]
        + [pltpu.SemaphoreType.DMA] * 6
    ),
)(_edge_body)


def _comb_body(p_ref, o_ref):
    o_ref[...] = jnp.tanh(p_ref[0] + p_ref[1])


def _combine(partials):
    return pl.pallas_call(
        _comb_body,
        grid=(N // MM_BLK,),
        in_specs=[pl.BlockSpec((NC, MM_BLK, D), lambda i: (0, i, 0))],
        out_specs=pl.BlockSpec((MM_BLK, D), lambda i: (i, 0)),
        out_shape=jax.ShapeDtypeStruct((N, D), jnp.float32),
    )(partials)


def kernel(x, edge_index, edge_weight, W, b):
    dst = edge_index[0].astype(jnp.int32)
    src = edge_index[1].astype(jnp.int32)
    npad = EPAD - E
    pad_idx = jnp.arange(npad, dtype=jnp.int32) % N
    src1 = jnp.concatenate([src, pad_idx])
    dst1 = jnp.concatenate([dst, pad_idx])
    ew1 = jnp.concatenate([edge_weight, jnp.zeros((npad,), jnp.float32)])
    support = _support_matmul(x, W, b.reshape(1, D))
    zeros = jnp.zeros((ROWS_PER_TILE, D), jnp.float32)
    partials = _edge_kernel(support, src1, dst1, ew1, zeros)
    return _combine(partials)
